# baseline (speedup 1.0000x reference)
import numpy as np
import jax
import jax.numpy as jnp
from jax import lax
from jax.experimental import pallas as pl
from jax.experimental.pallas import tpu as pltpu
from jax.experimental.pallas import tpu_sc as plsc

N = 16384
D = 512
C = 8
NC = 2
NS = 16
NW = NC * NS
ROWS_PER_WORKER = N // NW
CHUNK = 64
NCHUNK = ROWS_PER_WORKER // CHUNK
NBUF = 3
TOPK = 64
CF_ITERS = 64


def _sc_body(hid_hbm, ids_hbm, s_out, q_out, xbuf, idbuf, acc_s, acc_q, *sems):
    cid = lax.axis_index("c")
    sid = lax.axis_index("s")
    wid = cid * NS + sid
    base = wid * ROWS_PER_WORKER

    pltpu.sync_copy(ids_hbm.at[pl.ds(base, ROWS_PER_WORKER)],
                    idbuf.at[pl.ds(0, ROWS_PER_WORKER)])

    zero = jnp.zeros((16,), jnp.float32)
    for r in range(C):
        for k in range(D // 16):
            acc_s[r, pl.ds(k * 16, 16)] = zero
            acc_q[r, pl.ds(k * 16, 16)] = zero

    descs = [None] * NBUF
    for b in range(NBUF):
        descs[b] = pltpu.async_copy(
            hid_hbm.at[pl.ds(base + b * CHUNK, CHUNK)], xbuf.at[b], sems[b])

    for j in range(NCHUNK):
        b = j % NBUF
        descs[b].wait()

        def _row(r, carry):
            cvec = idbuf[pl.ds(j * CHUNK + r, 16)]
            c_sc = cvec[0]

            @plsc.parallel_loop(0, D // 16, 1, unroll=8)
            def _col(k):
                v = xbuf[b, r, pl.ds(k * 16, 16)]
                plsc.addupdate(acc_s.at[c_sc, pl.ds(k * 16, 16)], v)
                plsc.addupdate(acc_q.at[c_sc, pl.ds(k * 16, 16)], v * v)

            return carry

        lax.fori_loop(0, CHUNK, _row, 0)

        if j + NBUF < NCHUNK:
            descs[b] = pltpu.async_copy(
                hid_hbm.at[pl.ds(base + (j + NBUF) * CHUNK, CHUNK)],
                xbuf.at[b], sems[b])

    pltpu.sync_copy(acc_s, s_out.at[wid])
    pltpu.sync_copy(acc_q, q_out.at[wid])


def _sc_segment_sums(hidden, ids2):
    mesh = plsc.VectorSubcoreMesh(core_axis_name="c", subcore_axis_name="s",
                                  num_cores=NC, num_subcores=NS)
    return pl.kernel(
        _sc_body,
        out_type=(jax.ShapeDtypeStruct((NW, C, D), jnp.float32),
                  jax.ShapeDtypeStruct((NW, C, D), jnp.float32)),
        mesh=mesh,
        compiler_params=pltpu.CompilerParams(use_tc_tiling_on_sc=True,
                                             needs_layout_passes=False),
        scratch_types=(
            pltpu.VMEM((NBUF, CHUNK, D), jnp.float32),
            pltpu.VMEM((ROWS_PER_WORKER + 16,), jnp.int32),
            pltpu.VMEM((C, D), jnp.float32),
            pltpu.VMEM((C, D), jnp.float32),
        ) + (pltpu.SemaphoreType.DMA,) * NBUF,
    )(hidden, ids2)


def _ln_gamma_ratio(b):
    small = b < 8.0
    z = b + jnp.where(small, 8.0, 0.0)
    iz = 1.0 / z
    poly = 1.0 + iz * (-1.0 / 8.0 + iz * (1.0 / 128.0 + iz * (5.0 / 1024.0 + iz * (-21.0 / 32768.0))))
    lr = -(0.5 * jnp.log(z) + jnp.log(poly))
    corr = jnp.zeros_like(b)
    for i in range(8):
        corr = corr + jnp.where(small, jnp.log((b + i + 0.5) / (b + i)), 0.0)
    return lr + corr


def _betacf(a, b, x):
    qab = a + b
    qap = a + 1.0
    qam = a - 1.0
    tiny = 1e-30

    c0 = jnp.ones_like(x)
    d0 = 1.0 - qab * x / qap
    d0 = jnp.where(jnp.abs(d0) < tiny, tiny, d0)
    d0 = 1.0 / d0
    h0 = d0

    def body(m, carry):
        c, d, h = carry
        mf = m.astype(jnp.float32)
        m2 = 2.0 * mf
        aa = mf * (b - mf) * x / ((qam + m2) * (a + m2))
        d = 1.0 + aa * d
        d = jnp.where(jnp.abs(d) < tiny, tiny, d)
        c = 1.0 + aa / c
        c = jnp.where(jnp.abs(c) < tiny, tiny, c)
        d = 1.0 / d
        h = h * d * c
        aa = -(a + mf) * (qab + mf) * x / ((a + m2) * (qap + m2))
        d = 1.0 + aa * d
        d = jnp.where(jnp.abs(d) < tiny, tiny, d)
        c = 1.0 + aa / c
        c = jnp.where(jnp.abs(c) < tiny, tiny, c)
        d = 1.0 / d
        h = h * d * c
        return c, d, h

    _, _, h = lax.fori_loop(1, CF_ITERS + 1, body, (c0, d0, h0))
    return h


def _betainc_half(b, x):
    a = jnp.full_like(x, 0.5)
    bb = jnp.broadcast_to(b, x.shape).astype(jnp.float32)
    ln_b_fn = 0.5 * jnp.log(jnp.float32(np.pi)) + _ln_gamma_ratio(bb)
    ln_front = a * jnp.log(x) + bb * jnp.log1p(-x) - ln_b_fn
    front = jnp.exp(ln_front)
    use_direct = x < (a + 1.0) / (a + bb + 2.0)
    cf_dir = _betacf(a, bb, x)
    cf_sym = _betacf(bb, a, 1.0 - x)
    return jnp.where(use_direct, front * cf_dir / a, 1.0 - front * cf_sym / bb)


def _epilogue(S, Q, cnt, d_f):
    m = S / cnt
    W = Q - S * S / cnt
    ii, jj = np.triu_indices(C, k=1)
    ml = jnp.concatenate([m[i:i + 1] for i in ii], axis=0)
    mr = jnp.concatenate([m[j:j + 1] for j in jj], axis=0)
    Wp = (jnp.concatenate([W[i:i + 1] for i in ii], axis=0)
          + jnp.concatenate([W[j:j + 1] for j in jj], axis=0))
    cl = jnp.concatenate([cnt[i:i + 1] for i in ii], axis=0)
    cr = jnp.concatenate([cnt[j:j + 1] for j in jj], axis=0)

    B = (ml - mr) ** 2 * (cl + cr) * 0.25
    x = B / (B + Wp)
    xl = jnp.clip(x, 1e-37, 1.0 - 1e-5)
    d2 = cl + cr - 2.0
    d2 = jnp.where(d2 == 0.0, d2 + 1e-5, d2)
    b = d2 * 0.5

    P = xl.shape[0]
    colio = lax.broadcasted_iota(jnp.int32, (P, TOPK), 1)

    def extract(i, carry):
        xc, tot, Mbuf, Takebuf = carry
        mi = jnp.max(xc, axis=1, keepdims=True)
        eqm = xc == mi
        c = jnp.sum(eqm.astype(jnp.float32), axis=1, keepdims=True)
        xc = jnp.where(eqm, -1.0, xc)
        take = jnp.clip(d_f - tot, 0.0, c)
        col = colio == i
        Mbuf = jnp.where(col, mi, Mbuf)
        Takebuf = jnp.where(col, take, Takebuf)
        return xc, tot + c, Mbuf, Takebuf

    carry0 = (xl, jnp.zeros((P, 1), jnp.float32),
              jnp.zeros((P, TOPK), jnp.float32), jnp.zeros((P, TOPK), jnp.float32))
    _, _, Mbuf, Takebuf = lax.fori_loop(0, TOPK, extract, carry0)

    Mclean = jnp.clip(Mbuf, 1e-37, 1.0 - 1e-5)
    I = _betainc_half(b, Mclean)
    return -jnp.sum(Takebuf * jnp.log(I))


def _epi_kernel(s2_ref, q2_ref, ids_ref, d_ref, out_ref):
    S = s2_ref[0]
    Q = q2_ref[0]
    for w in range(1, NW):
        S = S + s2_ref[w]
        Q = Q + q2_ref[w]
    ids = ids_ref[...]
    cnt_rows = []
    for c in range(C):
        cnt_rows.append(jnp.full((1, 1), jnp.sum((ids == c).astype(jnp.float32))))
    cnt = jnp.concatenate(cnt_rows, axis=0)
    loss = _epilogue(S, Q, cnt, d_ref[0, 0])
    out_ref[...] = jnp.broadcast_to(loss, (1, 1))


def _tc_epilogue(S2, Q2, ids2, d):
    d_arr = jnp.full((1, 128), d, dtype=jnp.float32)
    out = pl.pallas_call(
        _epi_kernel,
        out_shape=jax.ShapeDtypeStruct((1, 1), jnp.float32),
    )(S2, Q2, ids2, d_arr)
    return out[0, 0]


def kernel(hidden, batch_ids, d, epoch, numEpoch, count_batch):
    ids1 = batch_ids.astype(jnp.int32)
    S2, Q2 = _sc_segment_sums(hidden, ids1)
    ids2 = ids1.reshape(N // CHUNK, CHUNK)
    return _tc_epilogue(S2, Q2, ids2, d)


# speedup vs baseline: 1.4668x; 1.4668x over previous
import numpy as np
import jax
import jax.numpy as jnp
from jax import lax
from jax.experimental import pallas as pl
from jax.experimental.pallas import tpu as pltpu
from jax.experimental.pallas import tpu_sc as plsc

N = 16384
D = 512
C = 8
NC = 2
NS = 16
NW = NC * NS
N_SC = 4096
N_TC = N - N_SC
ROWS_PER_WORKER = N_SC // NW
CHUNK = 64
NCHUNK = ROWS_PER_WORKER // CHUNK
NBUF = 2
TC_BLOCK = 2048
NBT = N_TC // TC_BLOCK
TOPK = 64
CF_ITERS = 64


def _sc_body(hid_hbm, ids_hbm, s_out, q_out, xbuf, idbuf, acc_s, acc_q, *sems):
    cid = lax.axis_index("c")
    sid = lax.axis_index("s")
    wid = cid * NS + sid
    base = wid * ROWS_PER_WORKER

    pltpu.sync_copy(ids_hbm.at[pl.ds(base, ROWS_PER_WORKER)],
                    idbuf.at[pl.ds(0, ROWS_PER_WORKER)])

    zero = jnp.zeros((16,), jnp.float32)
    for r in range(C):
        for k in range(D // 16):
            acc_s[r, pl.ds(k * 16, 16)] = zero
            acc_q[r, pl.ds(k * 16, 16)] = zero

    descs = [None] * NBUF
    for b in range(NBUF):
        descs[b] = pltpu.async_copy(
            hid_hbm.at[pl.ds(base + b * CHUNK, CHUNK)], xbuf.at[b], sems[b])

    for j in range(NCHUNK):
        b = j % NBUF
        descs[b].wait()

        def _row(r, carry):
            cvec = idbuf[pl.ds(j * CHUNK + r, 16)]
            c_sc = cvec[0]

            @plsc.parallel_loop(0, D // 16, 1, unroll=8)
            def _col(k):
                v = xbuf[b, r, pl.ds(k * 16, 16)]
                plsc.addupdate(acc_s.at[c_sc, pl.ds(k * 16, 16)], v)
                plsc.addupdate(acc_q.at[c_sc, pl.ds(k * 16, 16)], v * v)

            return carry

        lax.fori_loop(0, CHUNK, _row, 0)

        if j + NBUF < NCHUNK:
            descs[b] = pltpu.async_copy(
                hid_hbm.at[pl.ds(base + (j + NBUF) * CHUNK, CHUNK)],
                xbuf.at[b], sems[b])

    pltpu.sync_copy(acc_s, s_out.at[wid])
    pltpu.sync_copy(acc_q, q_out.at[wid])


def _sc_segment_sums(hidden, ids2):
    mesh = plsc.VectorSubcoreMesh(core_axis_name="c", subcore_axis_name="s",
                                  num_cores=NC, num_subcores=NS)
    return pl.kernel(
        _sc_body,
        out_type=(jax.ShapeDtypeStruct((NW, C, D), jnp.float32),
                  jax.ShapeDtypeStruct((NW, C, D), jnp.float32)),
        mesh=mesh,
        compiler_params=pltpu.CompilerParams(use_tc_tiling_on_sc=True,
                                             needs_layout_passes=False),
        scratch_types=(
            pltpu.VMEM((NBUF, CHUNK, D), jnp.float32),
            pltpu.VMEM((ROWS_PER_WORKER + 16,), jnp.int32),
            pltpu.VMEM((C, D), jnp.float32),
            pltpu.VMEM((C, D), jnp.float32),
        ) + (pltpu.SemaphoreType.DMA,) * NBUF,
    )(hidden, ids2)


def _ln_gamma_ratio(b):
    small = b < 8.0
    z = b + jnp.where(small, 8.0, 0.0)
    iz = 1.0 / z
    poly = 1.0 + iz * (-1.0 / 8.0 + iz * (1.0 / 128.0 + iz * (5.0 / 1024.0 + iz * (-21.0 / 32768.0))))
    lr = -(0.5 * jnp.log(z) + jnp.log(poly))
    corr = jnp.zeros_like(b)
    for i in range(8):
        corr = corr + jnp.where(small, jnp.log((b + i + 0.5) / (b + i)), 0.0)
    return lr + corr


def _betacf(a, b, x):
    qab = a + b
    qap = a + 1.0
    qam = a - 1.0
    tiny = 1e-30

    c0 = jnp.ones_like(x)
    d0 = 1.0 - qab * x / qap
    d0 = jnp.where(jnp.abs(d0) < tiny, tiny, d0)
    d0 = 1.0 / d0
    h0 = d0

    def body(m, carry):
        c, d, h = carry
        mf = m.astype(jnp.float32)
        m2 = 2.0 * mf
        aa = mf * (b - mf) * x / ((qam + m2) * (a + m2))
        d = 1.0 + aa * d
        d = jnp.where(jnp.abs(d) < tiny, tiny, d)
        c = 1.0 + aa / c
        c = jnp.where(jnp.abs(c) < tiny, tiny, c)
        d = 1.0 / d
        h = h * d * c
        aa = -(a + mf) * (qab + mf) * x / ((a + m2) * (qap + m2))
        d = 1.0 + aa * d
        d = jnp.where(jnp.abs(d) < tiny, tiny, d)
        c = 1.0 + aa / c
        c = jnp.where(jnp.abs(c) < tiny, tiny, c)
        d = 1.0 / d
        h = h * d * c
        return c, d, h

    _, _, h = lax.fori_loop(1, CF_ITERS + 1, body, (c0, d0, h0))
    return h


def _betainc_half(b, x):
    a = jnp.full_like(x, 0.5)
    bb = jnp.broadcast_to(b, x.shape).astype(jnp.float32)
    ln_b_fn = 0.5 * jnp.log(jnp.float32(np.pi)) + _ln_gamma_ratio(bb)
    ln_front = a * jnp.log(x) + bb * jnp.log1p(-x) - ln_b_fn
    front = jnp.exp(ln_front)
    use_direct = x < (a + 1.0) / (a + bb + 2.0)
    cf_dir = _betacf(a, bb, x)
    cf_sym = _betacf(bb, a, 1.0 - x)
    return jnp.where(use_direct, front * cf_dir / a, 1.0 - front * cf_sym / bb)


def _epilogue(S, Q, cnt, d_f):
    m = S / cnt
    W = Q - S * S / cnt
    ii, jj = np.triu_indices(C, k=1)
    ml = jnp.concatenate([m[i:i + 1] for i in ii], axis=0)
    mr = jnp.concatenate([m[j:j + 1] for j in jj], axis=0)
    Wp = (jnp.concatenate([W[i:i + 1] for i in ii], axis=0)
          + jnp.concatenate([W[j:j + 1] for j in jj], axis=0))
    cl = jnp.concatenate([cnt[i:i + 1] for i in ii], axis=0)
    cr = jnp.concatenate([cnt[j:j + 1] for j in jj], axis=0)

    B = (ml - mr) ** 2 * (cl + cr) * 0.25
    x = B / (B + Wp)
    xl = jnp.clip(x, 1e-37, 1.0 - 1e-5)
    d2 = cl + cr - 2.0
    d2 = jnp.where(d2 == 0.0, d2 + 1e-5, d2)
    b = d2 * 0.5

    P = xl.shape[0]
    colio = lax.broadcasted_iota(jnp.int32, (P, TOPK), 1)

    def extract(i, carry):
        xc, tot, Mbuf, Takebuf = carry
        mi = jnp.max(xc, axis=1, keepdims=True)
        eqm = xc == mi
        c = jnp.sum(eqm.astype(jnp.float32), axis=1, keepdims=True)
        xc = jnp.where(eqm, -1.0, xc)
        take = jnp.clip(d_f - tot, 0.0, c)
        col = colio == i
        Mbuf = jnp.where(col, mi, Mbuf)
        Takebuf = jnp.where(col, take, Takebuf)
        return xc, tot + c, Mbuf, Takebuf

    carry0 = (xl, jnp.zeros((P, 1), jnp.float32),
              jnp.zeros((P, TOPK), jnp.float32), jnp.zeros((P, TOPK), jnp.float32))
    _, _, Mbuf, Takebuf = lax.fori_loop(0, TOPK, extract, carry0)

    Mclean = jnp.clip(Mbuf, 1e-37, 1.0 - 1e-5)
    I = _betainc_half(b, Mclean)
    return -jnp.sum(Takebuf * jnp.log(I))


def _tc_front(hid_ref, ids_ref, s_ref, q_ref):
    i = pl.program_id(0)

    @pl.when(i == 0)
    def _init():
        s_ref[...] = jnp.zeros_like(s_ref)
        q_ref[...] = jnp.zeros_like(q_ref)

    x = hid_ref[...]
    ids = ids_ref[0]
    cls = lax.broadcasted_iota(jnp.int32, (C, TC_BLOCK), 0)
    oh = (ids == cls).astype(jnp.float32)
    s_ref[...] += jnp.dot(oh, x, preferred_element_type=jnp.float32)
    q_ref[...] += jnp.dot(oh, x * x, preferred_element_type=jnp.float32)


def _tc_front_sums(hidden, ids3):
    return pl.pallas_call(
        _tc_front,
        grid=(NBT,),
        in_specs=[
            pl.BlockSpec((TC_BLOCK, D), lambda i: (i + N_SC // TC_BLOCK, 0)),
            pl.BlockSpec((1, 1, TC_BLOCK), lambda i: (i + N_SC // TC_BLOCK, 0, 0)),
        ],
        out_specs=(pl.BlockSpec((C, D), lambda i: (0, 0)),
                   pl.BlockSpec((C, D), lambda i: (0, 0))),
        out_shape=(jax.ShapeDtypeStruct((C, D), jnp.float32),
                   jax.ShapeDtypeStruct((C, D), jnp.float32)),
    )(hidden, ids3)


def _epi_kernel(s2_ref, q2_ref, stc_ref, qtc_ref, ids_ref, d_ref, out_ref):
    S = stc_ref[...]
    Q = qtc_ref[...]
    for w in range(NW):
        S = S + s2_ref[w]
        Q = Q + q2_ref[w]
    ids = ids_ref[...]
    cnt_rows = []
    for c in range(C):
        cnt_rows.append(jnp.full((1, 1), jnp.sum((ids == c).astype(jnp.float32))))
    cnt = jnp.concatenate(cnt_rows, axis=0)
    loss = _epilogue(S, Q, cnt, d_ref[0, 0])
    out_ref[...] = jnp.broadcast_to(loss, (1, 1))


def _tc_epilogue(S2, Q2, S_tc, Q_tc, ids2, d):
    d_arr = jnp.full((1, 128), d, dtype=jnp.float32)
    out = pl.pallas_call(
        _epi_kernel,
        out_shape=jax.ShapeDtypeStruct((1, 1), jnp.float32),
    )(S2, Q2, S_tc, Q_tc, ids2, d_arr)
    return out[0, 0]


def kernel(hidden, batch_ids, d, epoch, numEpoch, count_batch):
    ids1 = batch_ids.astype(jnp.int32)
    S2, Q2 = _sc_segment_sums(hidden, ids1)
    ids3 = ids1.reshape(N // TC_BLOCK, 1, TC_BLOCK)
    S_tc, Q_tc = _tc_front_sums(hidden, ids3)
    ids2 = ids1.reshape(128, 128)
    return _tc_epilogue(S2, Q2, S_tc, Q_tc, ids2, d)


# speedup vs baseline: 1.7195x; 1.1723x over previous
import numpy as np
import jax
import jax.numpy as jnp
from jax import lax
from jax.experimental import pallas as pl
from jax.experimental.pallas import tpu as pltpu
from jax.experimental.pallas import tpu_sc as plsc

N = 16384
D = 512
C = 8
NC = 2
NS = 16
NW = NC * NS
N_SC = 3072
N_TC = N - N_SC
ROWS_PER_WORKER = N_SC // NW
CHUNK = 48
NCHUNK = ROWS_PER_WORKER // CHUNK
NBUF = 2
TC_BLOCK = 1024
NBT = N_TC // TC_BLOCK
TOPK = 64
CF_ITERS = 24


def _sc_body(hid_hbm, ids_hbm, s_out, q_out, xbuf, idbuf, acc_s, acc_q, *sems):
    cid = lax.axis_index("c")
    sid = lax.axis_index("s")
    wid = cid * NS + sid
    base = wid * ROWS_PER_WORKER

    pltpu.sync_copy(ids_hbm.at[pl.ds(base, ROWS_PER_WORKER)],
                    idbuf.at[pl.ds(0, ROWS_PER_WORKER)])

    zero = jnp.zeros((16,), jnp.float32)
    for r in range(C):
        for k in range(D // 16):
            acc_s[r, pl.ds(k * 16, 16)] = zero
            acc_q[r, pl.ds(k * 16, 16)] = zero

    descs = [None] * NBUF
    for b in range(NBUF):
        descs[b] = pltpu.async_copy(
            hid_hbm.at[pl.ds(base + b * CHUNK, CHUNK)], xbuf.at[b], sems[b])

    for j in range(NCHUNK):
        b = j % NBUF
        descs[b].wait()

        def _row(r, carry):
            cvec = idbuf[pl.ds(j * CHUNK + r, 16)]
            c_sc = cvec[0]

            @plsc.parallel_loop(0, D // 16, 1, unroll=8)
            def _col(k):
                v = xbuf[b, r, pl.ds(k * 16, 16)]
                plsc.addupdate(acc_s.at[c_sc, pl.ds(k * 16, 16)], v)
                plsc.addupdate(acc_q.at[c_sc, pl.ds(k * 16, 16)], v * v)

            return carry

        lax.fori_loop(0, CHUNK, _row, 0)

        if j + NBUF < NCHUNK:
            descs[b] = pltpu.async_copy(
                hid_hbm.at[pl.ds(base + (j + NBUF) * CHUNK, CHUNK)],
                xbuf.at[b], sems[b])

    pltpu.sync_copy(acc_s, s_out.at[wid])
    pltpu.sync_copy(acc_q, q_out.at[wid])


def _sc_segment_sums(hidden, ids2):
    mesh = plsc.VectorSubcoreMesh(core_axis_name="c", subcore_axis_name="s",
                                  num_cores=NC, num_subcores=NS)
    return pl.kernel(
        _sc_body,
        out_type=(jax.ShapeDtypeStruct((NW, C, D), jnp.float32),
                  jax.ShapeDtypeStruct((NW, C, D), jnp.float32)),
        mesh=mesh,
        compiler_params=pltpu.CompilerParams(use_tc_tiling_on_sc=True,
                                             needs_layout_passes=False),
        scratch_types=(
            pltpu.VMEM((NBUF, CHUNK, D), jnp.float32),
            pltpu.VMEM((ROWS_PER_WORKER + 16,), jnp.int32),
            pltpu.VMEM((C, D), jnp.float32),
            pltpu.VMEM((C, D), jnp.float32),
        ) + (pltpu.SemaphoreType.DMA,) * NBUF,
    )(hidden, ids2)


def _ln_gamma_ratio(b):
    small = b < 8.0
    z = b + jnp.where(small, 8.0, 0.0)
    iz = 1.0 / z
    poly = 1.0 + iz * (-1.0 / 8.0 + iz * (1.0 / 128.0 + iz * (5.0 / 1024.0 + iz * (-21.0 / 32768.0))))
    lr = -(0.5 * jnp.log(z) + jnp.log(poly))
    corr = jnp.zeros_like(b)
    for i in range(8):
        corr = corr + jnp.where(small, jnp.log((b + i + 0.5) / (b + i)), 0.0)
    return lr + corr


def _betacf(a, b, x):
    qab = a + b
    qap = a + 1.0
    qam = a - 1.0
    tiny = 1e-30

    c0 = jnp.ones_like(x)
    d0 = 1.0 - qab * x / qap
    d0 = jnp.where(jnp.abs(d0) < tiny, tiny, d0)
    d0 = 1.0 / d0
    h0 = d0

    def body(m, carry):
        c, d, h = carry
        mf = m.astype(jnp.float32)
        m2 = 2.0 * mf
        aa = mf * (b - mf) * x / ((qam + m2) * (a + m2))
        d = 1.0 + aa * d
        d = jnp.where(jnp.abs(d) < tiny, tiny, d)
        c = 1.0 + aa / c
        c = jnp.where(jnp.abs(c) < tiny, tiny, c)
        d = 1.0 / d
        h = h * d * c
        aa = -(a + mf) * (qab + mf) * x / ((a + m2) * (qap + m2))
        d = 1.0 + aa * d
        d = jnp.where(jnp.abs(d) < tiny, tiny, d)
        c = 1.0 + aa / c
        c = jnp.where(jnp.abs(c) < tiny, tiny, c)
        d = 1.0 / d
        h = h * d * c
        return c, d, h

    _, _, h = lax.fori_loop(1, CF_ITERS + 1, body, (c0, d0, h0))
    return h


def _betainc_half(b, x):
    a = jnp.full_like(x, 0.5)
    bb = jnp.broadcast_to(b, x.shape).astype(jnp.float32)
    ln_b_fn = 0.5 * jnp.log(jnp.float32(np.pi)) + _ln_gamma_ratio(bb)
    ln_front = a * jnp.log(x) + bb * jnp.log1p(-x) - ln_b_fn
    front = jnp.exp(ln_front)
    use_direct = x < (a + 1.0) / (a + bb + 2.0)
    aa_ = jnp.where(use_direct, a, bb)
    bb_ = jnp.where(use_direct, bb, a)
    xx_ = jnp.where(use_direct, x, 1.0 - x)
    cf = _betacf(aa_, bb_, xx_)
    return jnp.where(use_direct, front * cf / a, 1.0 - front * cf / bb)


def _epilogue(S, Q, cnt, d_f):
    m = S / cnt
    W = Q - S * S / cnt
    ii, jj = np.triu_indices(C, k=1)
    ml = jnp.concatenate([m[i:i + 1] for i in ii], axis=0)
    mr = jnp.concatenate([m[j:j + 1] for j in jj], axis=0)
    Wp = (jnp.concatenate([W[i:i + 1] for i in ii], axis=0)
          + jnp.concatenate([W[j:j + 1] for j in jj], axis=0))
    cl = jnp.concatenate([cnt[i:i + 1] for i in ii], axis=0)
    cr = jnp.concatenate([cnt[j:j + 1] for j in jj], axis=0)

    B = (ml - mr) ** 2 * (cl + cr) * 0.25
    x = B / (B + Wp)
    xl = jnp.clip(x, 1e-37, 1.0 - 1e-5)
    d2 = cl + cr - 2.0
    d2 = jnp.where(d2 == 0.0, d2 + 1e-5, d2)
    b = d2 * 0.5

    P = xl.shape[0]
    colio = lax.broadcasted_iota(jnp.int32, (P, TOPK), 1)

    def extract(i, carry):
        xc, tot, Mbuf, Takebuf = carry
        mi = jnp.max(xc, axis=1, keepdims=True)
        eqm = xc == mi
        c = jnp.sum(eqm.astype(jnp.float32), axis=1, keepdims=True)
        xc = jnp.where(eqm, -1.0, xc)
        take = jnp.clip(d_f - tot, 0.0, c)
        col = colio == i
        Mbuf = jnp.where(col, mi, Mbuf)
        Takebuf = jnp.where(col, take, Takebuf)
        return xc, tot + c, Mbuf, Takebuf

    carry0 = (xl, jnp.zeros((P, 1), jnp.float32),
              jnp.zeros((P, TOPK), jnp.float32), jnp.zeros((P, TOPK), jnp.float32))
    _, _, Mbuf, Takebuf = lax.fori_loop(0, TOPK, extract, carry0)

    Mclean = jnp.clip(Mbuf, 1e-37, 1.0 - 1e-5)
    I = _betainc_half(b, Mclean)
    return -jnp.sum(Takebuf * jnp.log(I))


def _tc_front(hid_ref, ids_ref, s_ref, q_ref):
    i = pl.program_id(0)

    @pl.when(i == 0)
    def _init():
        s_ref[...] = jnp.zeros_like(s_ref)
        q_ref[...] = jnp.zeros_like(q_ref)

    x = hid_ref[...]
    ids = ids_ref[0]
    cls = lax.broadcasted_iota(jnp.int32, (C, TC_BLOCK), 0)
    oh = (ids == cls).astype(jnp.float32)
    s_ref[...] += jnp.dot(oh, x, preferred_element_type=jnp.float32)
    q_ref[...] += jnp.dot(oh, x * x, preferred_element_type=jnp.float32)


def _tc_front_sums(hidden, ids3):
    return pl.pallas_call(
        _tc_front,
        grid=(NBT,),
        in_specs=[
            pl.BlockSpec((TC_BLOCK, D), lambda i: (i + N_SC // TC_BLOCK, 0)),
            pl.BlockSpec((1, 1, TC_BLOCK), lambda i: (i + N_SC // TC_BLOCK, 0, 0)),
        ],
        out_specs=(pl.BlockSpec((C, D), lambda i: (0, 0)),
                   pl.BlockSpec((C, D), lambda i: (0, 0))),
        out_shape=(jax.ShapeDtypeStruct((C, D), jnp.float32),
                   jax.ShapeDtypeStruct((C, D), jnp.float32)),
    )(hidden, ids3)


def _epi_kernel(s2_ref, q2_ref, stc_ref, qtc_ref, ids_ref, d_ref, out_ref):
    S = stc_ref[...]
    Q = qtc_ref[...]
    for w in range(NW):
        S = S + s2_ref[w]
        Q = Q + q2_ref[w]
    ids = ids_ref[...]
    cnt_rows = []
    for c in range(C):
        cnt_rows.append(jnp.full((1, 1), jnp.sum((ids == c).astype(jnp.float32))))
    cnt = jnp.concatenate(cnt_rows, axis=0)
    loss = _epilogue(S, Q, cnt, d_ref[0, 0])
    out_ref[...] = jnp.broadcast_to(loss, (1, 1))


def _tc_epilogue(S2, Q2, S_tc, Q_tc, ids2, d):
    d_arr = jnp.full((1, 128), d, dtype=jnp.float32)
    out = pl.pallas_call(
        _epi_kernel,
        out_shape=jax.ShapeDtypeStruct((1, 1), jnp.float32),
    )(S2, Q2, S_tc, Q_tc, ids2, d_arr)
    return out[0, 0]


def kernel(hidden, batch_ids, d, epoch, numEpoch, count_batch):
    ids1 = batch_ids.astype(jnp.int32)
    S2, Q2 = _sc_segment_sums(hidden, ids1)
    ids3 = ids1.reshape(N // TC_BLOCK, 1, TC_BLOCK)
    S_tc, Q_tc = _tc_front_sums(hidden, ids3)
    ids2 = ids1.reshape(128, 128)
    return _tc_epilogue(S2, Q2, S_tc, Q_tc, ids2, d)
